# Initial kernel scaffold; baseline (speedup 1.0000x reference)
import jax
import jax.numpy as jnp
from jax.experimental import pallas as pl


def kernel(x, w_gating, w1, w2):
    raise NotImplementedError("write your pallas kernel here")


# speedup vs baseline: 1.9184x; 1.9184x over previous
import functools

import jax
import jax.numpy as jnp
from jax import lax
from jax.experimental import pallas as pl
from jax.experimental.pallas import tpu as pltpu

_THR = 0.8
_CAP_FACTOR = 2.0
_MIN_CAP = 4
_LOSS_COEF = 0.01

_SORT_NET = [
    (0, 1), (2, 3), (4, 5), (6, 7),
    (0, 2), (1, 3), (4, 6), (5, 7),
    (1, 2), (5, 6),
    (0, 4), (1, 5), (2, 6), (3, 7),
    (2, 4), (3, 5),
    (1, 2), (3, 4), (5, 6),
]


def _cumsum_rows(a, n):
    k = 1
    while k < n:
        z = jnp.zeros((k,) + a.shape[1:], a.dtype)
        a = a + jnp.concatenate([z, a[:-k]], axis=0)
        k *= 2
    return a


def _gating_body(x_ref, wg_ref, wv_ref, valid_ref, pos_ref, loss_ref,
                 *, b, n, e, cap):
    x = x_ref[...]
    wg = wg_ref[...]
    logits = jnp.dot(x, wg, preferred_element_type=jnp.float32)
    m = jnp.max(logits, axis=1, keepdims=True)
    ex = jnp.exp(logits - m)
    p = ex / jnp.sum(ex, axis=1, keepdims=True)

    s = [p[:, i:i + 1] for i in range(e)]
    for (i, j) in _SORT_NET:
        lo = jnp.minimum(s[i], s[j])
        hi = jnp.maximum(s[i], s[j])
        s[i], s[j] = lo, hi
    sd = s[::-1]

    t = sd[0]
    k = jnp.ones_like(t)
    for i in range(e - 1):
        k = k + (t < _THR).astype(jnp.float32)
        t = t + sd[i + 1]
    max_k = jnp.max(k)
    k_eff = jnp.minimum(k, max_k)

    tsel = jnp.zeros_like(k)
    for i in range(e):
        tsel = tsel + sd[i] * (k_eff == float(i + 1)).astype(jnp.float32)

    mask = (p >= tsel).astype(jnp.float32)
    selp = p * mask
    norm = selp / jnp.sum(selp, axis=1, keepdims=True)

    incl = jnp.concatenate(
        [_cumsum_rows(mask[i * n:(i + 1) * n], n) for i in range(b)], axis=0)
    excl = incl - mask
    pie = excl * mask
    mask2 = mask * (pie < float(cap)).astype(jnp.float32)
    pie2 = pie * mask2
    pos = jnp.sum(pie2, axis=1, keepdims=True)
    valid = mask2 * (pos < float(cap)).astype(jnp.float32)
    wv = norm * valid

    acc = jnp.zeros((1, e), jnp.float32)
    for i in range(b):
        dens = jnp.sum(mask2[i * n:(i + 1) * n], axis=0, keepdims=True) / n
        prox = jnp.sum(p[i * n:(i + 1) * n], axis=0, keepdims=True) / n
        acc = acc + dens * prox
    loss = jnp.sum(acc) / float(b * e) * float(e * e) * _LOSS_COEF

    wv_ref[...] = wv
    valid_ref[...] = valid
    pos_ref[...] = pos
    loss_ref[...] = jnp.full((1, 1), loss, jnp.float32)


def _ffn_body(xb_ref, posr_ref, posc_ref, vrow_ref, wcol_ref, w1_ref, w2_ref,
              out_ref, ei_ref, acc_ref, *, n, cap, hc):
    eid = pl.program_id(1)
    hid = pl.program_id(2)

    @pl.when(hid == 0)
    def _():
        posr = posr_ref[0].astype(jnp.int32)
        vrow = vrow_ref[0, 0]
        ioc = lax.broadcasted_iota(jnp.int32, (cap, n), 0)
        oh = ((ioc == posr).astype(jnp.float32) * vrow).astype(jnp.bfloat16)
        ei_ref[...] = jnp.dot(oh, xb_ref[0],
                              preferred_element_type=jnp.float32)

    hidden = jnp.dot(ei_ref[...].astype(jnp.bfloat16), w1_ref[0],
                     preferred_element_type=jnp.float32)
    g = 0.5 * hidden * (1.0 + lax.erf(hidden * 0.7071067811865476))
    part = jnp.dot(g.astype(jnp.bfloat16), w2_ref[0],
                   preferred_element_type=jnp.float32)

    @pl.when(hid == 0)
    def _():
        acc_ref[...] = part

    @pl.when(hid > 0)
    def _():
        acc_ref[...] += part

    @pl.when(hid == hc - 1)
    def _():
        posc = posc_ref[0].astype(jnp.int32)
        wcol = wcol_ref[0, 0]
        ior = lax.broadcasted_iota(jnp.int32, (n, cap), 1)
        cmat = ((ior == posc).astype(jnp.float32) * wcol).astype(jnp.bfloat16)
        contrib = jnp.dot(cmat, acc_ref[...].astype(jnp.bfloat16),
                          preferred_element_type=jnp.float32)

        @pl.when(eid == 0)
        def _():
            out_ref[0] = contrib

        @pl.when(eid > 0)
        def _():
            out_ref[0] = out_ref[0] + contrib


def kernel(x, w_gating, w1, w2):
    b, n, d = x.shape
    e = w_gating.shape[-1]
    h = w1.shape[-1]
    assert e == 8
    cap = max(min(n, int(n * _CAP_FACTOR / e)), _MIN_CAP)
    bn = b * n

    xf = x.reshape(bn, d)
    gating = pl.pallas_call(
        functools.partial(_gating_body, b=b, n=n, e=e, cap=cap),
        out_shape=[
            jax.ShapeDtypeStruct((bn, e), jnp.float32),
            jax.ShapeDtypeStruct((bn, e), jnp.float32),
            jax.ShapeDtypeStruct((bn, 1), jnp.float32),
            jax.ShapeDtypeStruct((1, 1), jnp.float32),
        ],
    )
    wv, valid, pos, loss = gating(xf, w_gating)

    hblk = min(1024, h)
    hc = h // hblk

    xb16 = x.astype(jnp.bfloat16)
    w1b = w1.astype(jnp.bfloat16)
    w2b = w2.astype(jnp.bfloat16)
    pos_row = pos.reshape(b, 1, n)
    pos_col = pos.reshape(b, n, 1)
    v_row = valid.reshape(b, n, e).transpose(0, 2, 1).reshape(b, e, 1, n)
    w_col = wv.reshape(b, n, e).transpose(0, 2, 1).reshape(b, e, n, 1)

    out = pl.pallas_call(
        functools.partial(_ffn_body, n=n, cap=cap, hc=hc),
        grid=(b, e, hc),
        in_specs=[
            pl.BlockSpec((1, n, d), lambda i, j, k: (i, 0, 0)),
            pl.BlockSpec((1, 1, n), lambda i, j, k: (i, 0, 0)),
            pl.BlockSpec((1, n, 1), lambda i, j, k: (i, 0, 0)),
            pl.BlockSpec((1, 1, 1, n), lambda i, j, k: (i, j, 0, 0)),
            pl.BlockSpec((1, 1, n, 1), lambda i, j, k: (i, j, 0, 0)),
            pl.BlockSpec((1, d, hblk), lambda i, j, k: (j, 0, k)),
            pl.BlockSpec((1, hblk, d), lambda i, j, k: (j, k, 0)),
        ],
        out_specs=pl.BlockSpec((1, n, d), lambda i, j, k: (i, 0, 0)),
        out_shape=jax.ShapeDtypeStruct((b, n, d), jnp.float32),
        scratch_shapes=[
            pltpu.VMEM((cap, d), jnp.float32),
            pltpu.VMEM((cap, d), jnp.float32),
        ],
        compiler_params=pltpu.CompilerParams(
            dimension_semantics=("parallel", "arbitrary", "arbitrary")),
    )(xb16, pos_row, pos_col, v_row, w_col, w1b, w2b)

    return out, loss.reshape(())
